# baseline (speedup 1.0000x reference)
import jax
import jax.numpy as jnp
from jax import lax
from jax.experimental import pallas as pl
from jax.experimental.pallas import tpu as pltpu
from jax.experimental.pallas import tpu_sc as plsc

BATCH = 16384
EMBED = 64
L = 16

_info = plsc.get_sparse_core_info()
NC, NS = _info.num_cores, _info.num_subcores
NW = NC * NS
BPW = BATCH // NW
HALF = BPW // 2
GROUPS = HALF // L


def _mf_kernel(user_hbm, item_hbm, ue_hbm, ie_hbm, ub_hbm, ib_hbm, out_hbm,
               uidx_v, iidx_v, uhalf_v, ihalf_v, urows_v, irows_v,
               ub_v, ib_v, out_v, sem_u, sem_i, sem_ub, sem_ib):
    wid = lax.axis_index("s") * NC + lax.axis_index("c")
    base = wid * BPW

    pltpu.sync_copy(user_hbm.at[pl.ds(base, BPW)], uidx_v)
    pltpu.sync_copy(item_hbm.at[pl.ds(base, BPW)], iidx_v)

    def half_body(i, carry):
        uhalf_v[pl.ds(i * L, L)] = uidx_v[pl.ds(i * L, L)] >> 1
        ihalf_v[pl.ds(i * L, L)] = iidx_v[pl.ds(i * L, L)] >> 1
        return carry

    lax.fori_loop(0, BPW // L, half_body, 0)

    cp_ub = pltpu.async_copy(ub_hbm.at[uidx_v], ub_v, sem_ub)
    cp_ib = pltpu.async_copy(ib_hbm.at[iidx_v], ib_v, sem_ib)

    lane = lax.iota(jnp.int32, L)

    for p in range(2):
        cp_u = pltpu.async_copy(
            ue_hbm.at[uhalf_v.at[pl.ds(p * HALF, HALF)]], urows_v, sem_u)
        cp_i = pltpu.async_copy(
            ie_hbm.at[ihalf_v.at[pl.ds(p * HALF, HALF)]], irows_v, sem_i)
        cp_u.wait()
        cp_i.wait()

        def group_body(g, carry):
            row0 = g * L
            rows16 = row0 + lane
            ucol0 = (uidx_v[pl.ds(p * HALF + row0, L)] & 1) * EMBED
            icol0 = (iidx_v[pl.ds(p * HALF + row0, L)] & 1) * EMBED
            acc = jnp.zeros((L,), jnp.float32)
            for e in range(EMBED):
                uv = plsc.load_gather(urows_v, [rows16, ucol0 + e])
                iv = plsc.load_gather(irows_v, [rows16, icol0 + e])
                acc = acc + uv * iv
            out_v[pl.ds(p * HALF + row0, L)] = acc
            return carry

        lax.fori_loop(0, GROUPS, group_body, 0)

    cp_ub.wait()
    cp_ib.wait()

    def bias_body(i, carry):
        sl = pl.ds(i * L, L)
        out_v[sl] = out_v[sl] + ub_v[sl] + ib_v[sl]
        return carry

    lax.fori_loop(0, BPW // L, bias_body, 0)

    pltpu.sync_copy(out_v, out_hbm.at[pl.ds(base, BPW)])


@jax.jit
def _mf(user, item, user_e2, item_e2, ub_flat, ib_flat):
    mesh = plsc.VectorSubcoreMesh(core_axis_name="c", subcore_axis_name="s")
    return pl.kernel(
        _mf_kernel,
        mesh=mesh,
        out_type=jax.ShapeDtypeStruct((BATCH,), jnp.float32),
        compiler_params=pltpu.CompilerParams(use_tc_tiling_on_sc=True,
                                             needs_layout_passes=False),
        scratch_types=[
            pltpu.VMEM((BPW,), jnp.int32),
            pltpu.VMEM((BPW,), jnp.int32),
            pltpu.VMEM((BPW,), jnp.int32),
            pltpu.VMEM((BPW,), jnp.int32),
            pltpu.VMEM((HALF, 2 * EMBED), jnp.float32),
            pltpu.VMEM((HALF, 2 * EMBED), jnp.float32),
            pltpu.VMEM((BPW,), jnp.float32),
            pltpu.VMEM((BPW,), jnp.float32),
            pltpu.VMEM((BPW,), jnp.float32),
            pltpu.SemaphoreType.DMA,
            pltpu.SemaphoreType.DMA,
            pltpu.SemaphoreType.DMA,
            pltpu.SemaphoreType.DMA,
        ],
    )(user, item, user_e2, item_e2, ub_flat, ib_flat)


def kernel(user, item, user_e, item_e, user_b, item_b):
    return _mf(user.astype(jnp.int32), item.astype(jnp.int32),
               user_e.reshape(-1, 2 * EMBED), item_e.reshape(-1, 2 * EMBED),
               user_b.reshape(-1), item_b.reshape(-1))


# speedup vs baseline: 2.1164x; 2.1164x over previous
import jax
import jax.numpy as jnp
from jax import lax
from jax.experimental import pallas as pl
from jax.experimental.pallas import tpu as pltpu
from jax.experimental.pallas import tpu_sc as plsc

BATCH = 16384
EMBED = 64
L = 16
TR = 8

_info = plsc.get_sparse_core_info()
NC, NS = _info.num_cores, _info.num_subcores
NW = NC * NS
BPW = BATCH // NW
CH = 32
CHUNKS = BPW // CH
CGROUPS = CH // L


def _mf_kernel(user_hbm, item_hbm, ue_hbm, ie_hbm, ub_hbm, ib_hbm, out_hbm,
               uidx_v, iidx_v, utid_v, itid_v, utiles_v, itiles_v,
               ub_v, ib_v, out_v, sem_u, sem_i, sem_ub, sem_ib):
    wid = lax.axis_index("s") * NC + lax.axis_index("c")
    base = wid * BPW

    pltpu.sync_copy(user_hbm.at[pl.ds(base, BPW)], uidx_v)
    pltpu.sync_copy(item_hbm.at[pl.ds(base, BPW)], iidx_v)

    def tid_body(i, carry):
        utid_v[pl.ds(i * L, L)] = uidx_v[pl.ds(i * L, L)] >> 3
        itid_v[pl.ds(i * L, L)] = iidx_v[pl.ds(i * L, L)] >> 3
        return carry

    lax.fori_loop(0, BPW // L, tid_body, 0)

    cp_ub = pltpu.async_copy(ub_hbm.at[uidx_v], ub_v, sem_ub)
    cp_ib = pltpu.async_copy(ib_hbm.at[iidx_v], ib_v, sem_ib)

    lane = lax.iota(jnp.int32, L)

    def chunk_body(c, carry):
        row0 = c * CH
        for g in range(CGROUPS):
            utid16 = utid_v[pl.ds(row0 + g * L, L)]
            itid16 = itid_v[pl.ds(row0 + g * L, L)]
            for j in range(L):
                slot = g * L + j
                pltpu.async_copy(
                    ue_hbm.at[pl.ds(utid16[j], 1)],
                    utiles_v.at[pl.ds(slot, 1)], sem_u)
                pltpu.async_copy(
                    ie_hbm.at[pl.ds(itid16[j], 1)],
                    itiles_v.at[pl.ds(slot, 1)], sem_i)
        for j in range(CH):
            pltpu.make_async_copy(
                ue_hbm.at[pl.ds(0, 1)],
                utiles_v.at[pl.ds(j, 1)], sem_u).wait()
            pltpu.make_async_copy(
                ie_hbm.at[pl.ds(0, 1)],
                itiles_v.at[pl.ds(j, 1)], sem_i).wait()
        for g in range(CGROUPS):
            tiles16 = g * L + lane
            ur16 = uidx_v[pl.ds(row0 + g * L, L)] & (TR - 1)
            ir16 = iidx_v[pl.ds(row0 + g * L, L)] & (TR - 1)
            acc = jnp.zeros((L,), jnp.float32)
            for e in range(EMBED):
                ecol = jnp.full((L,), e, jnp.int32)
                uv = plsc.load_gather(utiles_v, [tiles16, ur16, ecol])
                iv = plsc.load_gather(itiles_v, [tiles16, ir16, ecol])
                acc = acc + uv * iv
            out_v[pl.ds(row0 + g * L, L)] = acc
        return carry

    lax.fori_loop(0, CHUNKS, chunk_body, 0)

    cp_ub.wait()
    cp_ib.wait()

    def bias_body(i, carry):
        sl = pl.ds(i * L, L)
        out_v[sl] = out_v[sl] + ub_v[sl] + ib_v[sl]
        return carry

    lax.fori_loop(0, BPW // L, bias_body, 0)

    pltpu.sync_copy(out_v, out_hbm.at[pl.ds(base, BPW)])


@jax.jit
def _mf(user, item, user_e3, item_e3, ub_flat, ib_flat):
    mesh = plsc.VectorSubcoreMesh(core_axis_name="c", subcore_axis_name="s")
    return pl.kernel(
        _mf_kernel,
        mesh=mesh,
        out_type=jax.ShapeDtypeStruct((BATCH,), jnp.float32),
        compiler_params=pltpu.CompilerParams(use_tc_tiling_on_sc=True,
                                             needs_layout_passes=False),
        scratch_types=[
            pltpu.VMEM((BPW,), jnp.int32),
            pltpu.VMEM((BPW,), jnp.int32),
            pltpu.VMEM((BPW,), jnp.int32),
            pltpu.VMEM((BPW,), jnp.int32),
            pltpu.VMEM((CH, TR, EMBED), jnp.float32),
            pltpu.VMEM((CH, TR, EMBED), jnp.float32),
            pltpu.VMEM((BPW,), jnp.float32),
            pltpu.VMEM((BPW,), jnp.float32),
            pltpu.VMEM((BPW,), jnp.float32),
            pltpu.SemaphoreType.DMA,
            pltpu.SemaphoreType.DMA,
            pltpu.SemaphoreType.DMA,
            pltpu.SemaphoreType.DMA,
        ],
    )(user, item, user_e3, item_e3, ub_flat, ib_flat)


def kernel(user, item, user_e, item_e, user_b, item_b):
    return _mf(user.astype(jnp.int32), item.astype(jnp.int32),
               user_e.reshape(-1, TR, EMBED), item_e.reshape(-1, TR, EMBED),
               user_b.reshape(-1), item_b.reshape(-1))


# speedup vs baseline: 2.2558x; 1.0658x over previous
import jax
import jax.numpy as jnp
from jax import lax
from jax.experimental import pallas as pl
from jax.experimental.pallas import tpu as pltpu
from jax.experimental.pallas import tpu_sc as plsc

BATCH = 16384
EMBED = 64
L = 16
TR = 8

_info = plsc.get_sparse_core_info()
NC, NS = _info.num_cores, _info.num_subcores
NW = NC * NS
BPW = BATCH // NW
CH = 16
CHUNKS = BPW // CH
PAIRS = CHUNKS // 2


def _mf_kernel(user_hbm, item_hbm, ue_hbm, ie_hbm, ub_hbm, ib_hbm, out_hbm,
               uidx_v, iidx_v, utid_v, itid_v,
               ut0_v, ut1_v, it0_v, it1_v,
               ub_v, ib_v, out_v,
               sem_u0, sem_u1, sem_i0, sem_i1, sem_ub, sem_ib):
    wid = lax.axis_index("s") * NC + lax.axis_index("c")
    base = wid * BPW

    pltpu.sync_copy(user_hbm.at[pl.ds(base, BPW)], uidx_v)
    pltpu.sync_copy(item_hbm.at[pl.ds(base, BPW)], iidx_v)

    def tid_body(i, carry):
        utid_v[pl.ds(i * L, L)] = uidx_v[pl.ds(i * L, L)] >> 3
        itid_v[pl.ds(i * L, L)] = iidx_v[pl.ds(i * L, L)] >> 3
        return carry

    lax.fori_loop(0, BPW // L, tid_body, 0)

    cp_ub = pltpu.async_copy(ub_hbm.at[uidx_v], ub_v, sem_ub)
    cp_ib = pltpu.async_copy(ib_hbm.at[iidx_v], ib_v, sem_ib)

    lane = lax.iota(jnp.int32, L)

    def fire(c, ut_v, it_v, sem_u, sem_i):
        row0 = c * CH
        utid16 = utid_v[pl.ds(row0, L)]
        itid16 = itid_v[pl.ds(row0, L)]
        for j in range(CH):
            pltpu.async_copy(ue_hbm.at[pl.ds(utid16[j], 1)],
                             ut_v.at[pl.ds(j, 1)], sem_u)
            pltpu.async_copy(ie_hbm.at[pl.ds(itid16[j], 1)],
                             it_v.at[pl.ds(j, 1)], sem_i)

    def drain(ut_v, it_v, sem_u, sem_i):
        for j in range(CH):
            pltpu.make_async_copy(ue_hbm.at[pl.ds(0, 1)],
                                  ut_v.at[pl.ds(j, 1)], sem_u).wait()
            pltpu.make_async_copy(ie_hbm.at[pl.ds(0, 1)],
                                  it_v.at[pl.ds(j, 1)], sem_i).wait()

    def compute(c, ut_v, it_v):
        row0 = c * CH
        ur16 = uidx_v[pl.ds(row0, L)] & (TR - 1)
        ir16 = iidx_v[pl.ds(row0, L)] & (TR - 1)
        acc = jnp.zeros((L,), jnp.float32)
        for e in range(EMBED):
            ecol = jnp.full((L,), e, jnp.int32)
            uv = plsc.load_gather(ut_v, [lane, ur16, ecol])
            iv = plsc.load_gather(it_v, [lane, ir16, ecol])
            acc = acc + uv * iv
        out_v[pl.ds(row0, L)] = acc + ub_v[pl.ds(row0, L)] + ib_v[pl.ds(row0, L)]

    cp_ub.wait()
    cp_ib.wait()

    fire(0, ut0_v, it0_v, sem_u0, sem_i0)

    def pair_body(k, carry):
        c0 = 2 * k
        fire(c0 + 1, ut1_v, it1_v, sem_u1, sem_i1)
        drain(ut0_v, it0_v, sem_u0, sem_i0)
        compute(c0, ut0_v, it0_v)

        @pl.when(k < PAIRS - 1)
        def _():
            fire(c0 + 2, ut0_v, it0_v, sem_u0, sem_i0)

        drain(ut1_v, it1_v, sem_u1, sem_i1)
        compute(c0 + 1, ut1_v, it1_v)
        return carry

    lax.fori_loop(0, PAIRS, pair_body, 0)

    pltpu.sync_copy(out_v, out_hbm.at[pl.ds(base, BPW)])


@jax.jit
def _mf(user, item, user_e3, item_e3, ub_flat, ib_flat):
    mesh = plsc.VectorSubcoreMesh(core_axis_name="c", subcore_axis_name="s")
    tiles = pltpu.VMEM((CH, TR, EMBED), jnp.float32)
    return pl.kernel(
        _mf_kernel,
        mesh=mesh,
        out_type=jax.ShapeDtypeStruct((BATCH,), jnp.float32),
        compiler_params=pltpu.CompilerParams(use_tc_tiling_on_sc=True,
                                             needs_layout_passes=False),
        scratch_types=[
            pltpu.VMEM((BPW,), jnp.int32),
            pltpu.VMEM((BPW,), jnp.int32),
            pltpu.VMEM((BPW,), jnp.int32),
            pltpu.VMEM((BPW,), jnp.int32),
            tiles, tiles,
            tiles, tiles,
            pltpu.VMEM((BPW,), jnp.float32),
            pltpu.VMEM((BPW,), jnp.float32),
            pltpu.VMEM((BPW,), jnp.float32),
            pltpu.SemaphoreType.DMA,
            pltpu.SemaphoreType.DMA,
            pltpu.SemaphoreType.DMA,
            pltpu.SemaphoreType.DMA,
            pltpu.SemaphoreType.DMA,
            pltpu.SemaphoreType.DMA,
        ],
    )(user, item, user_e3, item_e3, ub_flat, ib_flat)


def kernel(user, item, user_e, item_e, user_b, item_b):
    return _mf(user.astype(jnp.int32), item.astype(jnp.int32),
               user_e.reshape(-1, TR, EMBED), item_e.reshape(-1, TR, EMBED),
               user_b.reshape(-1), item_b.reshape(-1))
